# baseline (speedup 1.0000x reference)
import jax
import jax.numpy as jnp
import numpy as np
from jax import lax
from jax.experimental import pallas as pl
from jax.experimental.pallas import tpu as pltpu
from jax.experimental.pallas import tpu_sc as plsc

N, C = 2, 32
X, Y, U, V = 16, 16, 32, 32
P = 262144
NVOX = X * Y * U * V
NC, NS = 2, 16
NW = NC * NS
PW = P // NW
BC = 64
NCHUNK = PW // BC
NT = N * NCHUNK
NG = BC // 16
NROW = 16 * BC
NDMA = NROW // 128
CB = 4 * BC
OB = BC * C


def _sc_body(vol, coords, out, coords_v, idx_v, w_v, rows_v, out_v,
             sem_c0, sem_c1, sem_g0, sem_g1, sem_o0, sem_o1):
    wid = lax.axis_index("s") * NC + lax.axis_index("c")
    sem_c = (sem_c0, sem_c1)
    sem_g = (sem_g0, sem_g1)
    sem_o = (sem_o0, sem_o1)

    def coff(t):
        return (((t >> 7) * NW + wid) * NCHUNK + (t & (NCHUNK - 1))) * CB

    def ooff(t):
        return ((t >> 7) * P + wid * PW + (t & (NCHUNK - 1)) * BC) * C

    def fire_coords(t, b):
        pltpu.async_copy(
            coords.at[pl.ds(coff(t), CB)],
            coords_v.at[pl.ds(b * CB, CB)],
            sem_c[b],
        )

    def fill(t, b):
        pltpu.make_async_copy(
            coords.at[pl.ds(0, CB)],
            coords_v.at[pl.ds(b * CB, CB)],
            sem_c[b],
        ).wait()
        nbase = (t >> 7) * NVOX
        for g in range(NG):
            cbb = b * CB
            xs = coords_v[pl.ds(cbb + 0 * BC + g * 16, 16)]
            ys = coords_v[pl.ds(cbb + 1 * BC + g * 16, 16)]
            us = coords_v[pl.ds(cbb + 2 * BC + g * 16, 16)]
            vs = coords_v[pl.ds(cbb + 3 * BC + g * 16, 16)]
            px = ((xs + 1.0) * 0.5) * float(X - 1)
            py = ((ys + 1.0) * 0.5) * float(Y - 1)
            pu = ((us + 1.0) * 0.5) * float(U - 1)
            pv = ((vs + 1.0) * 0.5) * float(V - 1)
            ix = jnp.clip(px.astype(jnp.int32), 0, X - 2)
            iy = jnp.clip(py.astype(jnp.int32), 0, Y - 2)
            iu = jnp.clip(pu.astype(jnp.int32), 0, U - 2)
            iv = jnp.clip(pv.astype(jnp.int32), 0, V - 2)
            fx = px - ix.astype(jnp.float32)
            fy = py - iy.astype(jnp.float32)
            fu = pu - iu.astype(jnp.float32)
            fv = pv - iv.astype(jnp.float32)
            base_i = (ix << 14) + (iy << 10) + (iu << 5) + iv + nbase
            wx = (1.0 - fx, fx)
            wy = (1.0 - fy, fy)
            wu = (1.0 - fu, fu)
            wv_ = (1.0 - fv, fv)
            wxy = [wx[a] * wy[bb] for bb in range(2) for a in range(2)]
            wuv = [wu[a] * wv_[bb] for bb in range(2) for a in range(2)]
            for k in range(16):
                dx, dy = (k >> 0) & 1, (k >> 1) & 1
                du, dv = (k >> 2) & 1, (k >> 3) & 1
                off = dx * 16384 + dy * 1024 + du * 32 + dv
                wk = wxy[dy * 2 + dx] * wuv[dv * 2 + du]
                pos = b * NROW + k * BC + g * 16
                idx_v[pl.ds(pos, 16)] = base_i + off
                w_v[pl.ds(pos, 16)] = wk
        for j in range(NDMA):
            pltpu.async_copy(
                vol.at[idx_v.at[pl.ds(b * NROW + j * 128, 128)]],
                rows_v.at[pl.ds(b * NROW + j * 128, 128), :],
                sem_g[b],
            )

        @pl.when(t + 2 < NT)
        def _():
            fire_coords(t + 2, b)

    def drain(t, b):
        @pl.when(t >= 2)
        def _():
            pltpu.make_async_copy(
                out_v.at[pl.ds(b * OB, OB)],
                out.at[pl.ds(0, OB)],
                sem_o[b],
            ).wait()

        pltpu.make_async_copy(
            vol.at[pl.ds(0, NROW), :],
            rows_v.at[pl.ds(b * NROW, NROW), :],
            sem_g[b],
        ).wait()

        def grp(g2, c2):
            gb = g2 * 16
            wv = [
                w_v[pl.ds(b * NROW + k * BC + gb, 16)] for k in range(16)
            ]
            def unpk(row):
                rv = rows_v[row, pl.ds(0, 16)]
                lo = lax.bitcast_convert_type(rv << 16, jnp.float32)
                hi = lax.bitcast_convert_type(
                    rv & jnp.int32(-65536), jnp.float32
                )
                return lo, hi

            for l in range(16):
                p = b * NROW + gb + l
                r0, r1 = unpk(p)
                a0 = wv[0][l] * r0
                a1 = wv[0][l] * r1
                for k in range(1, 16):
                    rk0, rk1 = unpk(k * BC + p)
                    a0 = a0 + wv[k][l] * rk0
                    a1 = a1 + wv[k][l] * rk1
                op = b * OB + (gb + l) * 32
                out_v[pl.ds(op, 16)] = a0
                out_v[pl.ds(op + 16, 16)] = a1
            return c2

        lax.fori_loop(0, NG, grp, jnp.int32(0))
        pltpu.async_copy(
            out_v.at[pl.ds(b * OB, OB)],
            out.at[pl.ds(ooff(t), OB)],
            sem_o[b],
        )

    fire_coords(jnp.int32(0), 0)
    fire_coords(jnp.int32(1), 1)
    fill(jnp.int32(0), 0)

    def pair(ip, carry):
        t0 = ip * 2
        fill(t0 + 1, 1)
        drain(t0, 0)

        @pl.when(t0 + 2 < NT)
        def _():
            fill(t0 + 2, 0)

        drain(t0 + 1, 1)
        return carry

    lax.fori_loop(0, NT // 2, pair, jnp.int32(0))
    for b in range(2):
        pltpu.make_async_copy(
            out_v.at[pl.ds(b * OB, OB)],
            out.at[pl.ds(0, OB)],
            sem_o[b],
        ).wait()


_mesh = plsc.VectorSubcoreMesh(
    core_axis_name="c", subcore_axis_name="s", num_cores=NC, num_subcores=NS
)

_run = pl.kernel(
    _sc_body,
    out_type=jax.ShapeDtypeStruct((N * P * C,), jnp.float32),
    mesh=_mesh,
    scratch_types=[
        pltpu.VMEM((2 * CB,), jnp.float32),
        pltpu.VMEM((2 * NROW,), jnp.int32),
        pltpu.VMEM((2 * NROW,), jnp.float32),
        pltpu.VMEM((2 * NROW, C // 2), jnp.int32),
        pltpu.VMEM((2 * OB,), jnp.float32),
        pltpu.SemaphoreType.DMA,
        pltpu.SemaphoreType.DMA,
        pltpu.SemaphoreType.DMA,
        pltpu.SemaphoreType.DMA,
        pltpu.SemaphoreType.DMA,
        pltpu.SemaphoreType.DMA,
    ],
    compiler_params=pltpu.CompilerParams(use_tc_tiling_on_sc=False),
)


def kernel(input, grid):
    vol = jax.lax.bitcast_convert_type(
        jnp.swapaxes(input.reshape(N, C, NVOX), 1, 2)
        .astype(jnp.bfloat16)
        .reshape(N * NVOX, C // 2, 2),
        jnp.int32,
    )
    coords = (
        grid.reshape(N, NW, NCHUNK, BC, 4)
        .transpose(0, 1, 2, 4, 3)
        .reshape(-1)
    )
    out_flat = _run(vol, coords)
    return (
        out_flat.reshape(N, P, 2, C // 2)
        .transpose(0, 3, 2, 1)
        .reshape(N, C, P)
    )


# speedup vs baseline: 1.5743x; 1.5743x over previous
import jax
import jax.numpy as jnp
import numpy as np
from jax import lax
from jax.experimental import pallas as pl
from jax.experimental.pallas import tpu as pltpu
from jax.experimental.pallas import tpu_sc as plsc

N, C = 2, 32
X, Y, U, V = 16, 16, 32, 32
P = 262144
NVOX = X * Y * U * V
NC, NS = 2, 16
NW = NC * NS
PW = P // NW
BC = 64
NCHUNK = PW // BC
NT = N * NCHUNK
NG = BC // 16
NROW = 16 * BC
NDMA = NROW // 128
CB = 4 * BC
OB = BC * C // 2


def _sc_body(vol, coords, out, coords_v, idx_v, w_v, rows_v, out_v,
             sem_c0, sem_c1, sem_g0, sem_g1, sem_o0, sem_o1):
    wid = lax.axis_index("s") * NC + lax.axis_index("c")
    sem_c = (sem_c0, sem_c1)
    sem_g = (sem_g0, sem_g1)
    sem_o = (sem_o0, sem_o1)

    def coff(t):
        return (((t >> 7) * NW + wid) * NCHUNK + (t & (NCHUNK - 1))) * CB

    def ooff(t):
        return ((t >> 7) * P + wid * PW + (t & (NCHUNK - 1)) * BC) * (C // 2)

    def fire_coords(t, b):
        pltpu.async_copy(
            coords.at[pl.ds(coff(t), CB)],
            coords_v.at[pl.ds(b * CB, CB)],
            sem_c[b],
        )

    def fill(t, b):
        pltpu.make_async_copy(
            coords.at[pl.ds(0, CB)],
            coords_v.at[pl.ds(b * CB, CB)],
            sem_c[b],
        ).wait()
        nbase = (t >> 7) * NVOX
        for g in range(NG):
            cbb = b * CB
            xs = coords_v[pl.ds(cbb + 0 * BC + g * 16, 16)]
            ys = coords_v[pl.ds(cbb + 1 * BC + g * 16, 16)]
            us = coords_v[pl.ds(cbb + 2 * BC + g * 16, 16)]
            vs = coords_v[pl.ds(cbb + 3 * BC + g * 16, 16)]
            px = ((xs + 1.0) * 0.5) * float(X - 1)
            py = ((ys + 1.0) * 0.5) * float(Y - 1)
            pu = ((us + 1.0) * 0.5) * float(U - 1)
            pv = ((vs + 1.0) * 0.5) * float(V - 1)
            ix = jnp.clip(px.astype(jnp.int32), 0, X - 2)
            iy = jnp.clip(py.astype(jnp.int32), 0, Y - 2)
            iu = jnp.clip(pu.astype(jnp.int32), 0, U - 2)
            iv = jnp.clip(pv.astype(jnp.int32), 0, V - 2)
            fx = px - ix.astype(jnp.float32)
            fy = py - iy.astype(jnp.float32)
            fu = pu - iu.astype(jnp.float32)
            fv = pv - iv.astype(jnp.float32)
            base_i = (ix << 14) + (iy << 10) + (iu << 5) + iv + nbase
            wx = (1.0 - fx, fx)
            wy = (1.0 - fy, fy)
            wu = (1.0 - fu, fu)
            wv_ = (1.0 - fv, fv)
            wxy = [wx[a] * wy[bb] for bb in range(2) for a in range(2)]
            wuv = [wu[a] * wv_[bb] for bb in range(2) for a in range(2)]
            for k in range(16):
                dx, dy = (k >> 0) & 1, (k >> 1) & 1
                du, dv = (k >> 2) & 1, (k >> 3) & 1
                off = dx * 16384 + dy * 1024 + du * 32 + dv
                wk = wxy[dy * 2 + dx] * wuv[dv * 2 + du]
                pos = b * NROW + k * BC + g * 16
                idx_v[pl.ds(pos, 16)] = base_i + off
                w_v[pl.ds(pos, 16)] = wk
        for j in range(NDMA):
            pltpu.async_copy(
                vol.at[idx_v.at[pl.ds(b * NROW + j * 128, 128)]],
                rows_v.at[pl.ds(b * NROW + j * 128, 128), :],
                sem_g[b],
            )

        @pl.when(t + 2 < NT)
        def _():
            fire_coords(t + 2, b)

    def drain(t, b):
        @pl.when(t >= 2)
        def _():
            pltpu.make_async_copy(
                out_v.at[pl.ds(b * OB, OB)],
                out.at[pl.ds(0, OB)],
                sem_o[b],
            ).wait()

        pltpu.make_async_copy(
            vol.at[pl.ds(0, NROW), :],
            rows_v.at[pl.ds(b * NROW, NROW), :],
            sem_g[b],
        ).wait()

        def grp(g2, c2):
            gb = g2 * 16
            wv = [
                w_v[pl.ds(b * NROW + k * BC + gb, 16)] for k in range(16)
            ]
            def unpk(row):
                rv = rows_v[row, pl.ds(0, 16)]
                lo = lax.bitcast_convert_type(rv << 16, jnp.float32)
                hi = lax.bitcast_convert_type(
                    rv & jnp.int32(-65536), jnp.float32
                )
                return lo, hi

            for l in range(16):
                p = b * NROW + gb + l
                r0, r1 = unpk(p)
                a0 = wv[0][l] * r0
                a1 = wv[0][l] * r1
                for k in range(1, 16):
                    rk0, rk1 = unpk(k * BC + p)
                    a0 = a0 + wv[k][l] * rk0
                    a1 = a1 + wv[k][l] * rk1
                u0 = lax.bitcast_convert_type(a0, jnp.int32)
                u1 = lax.bitcast_convert_type(a1, jnp.int32)
                r0 = u0 + (jnp.int32(32767) + ((u0 >> 16) & 1))
                r1 = u1 + (jnp.int32(32767) + ((u1 >> 16) & 1))
                lane = ((r0 >> 16) & jnp.int32(65535)) | (
                    r1 & jnp.int32(-65536)
                )
                out_v[pl.ds(b * OB + (gb + l) * 16, 16)] = lane
            return c2

        lax.fori_loop(0, NG, grp, jnp.int32(0))
        pltpu.async_copy(
            out_v.at[pl.ds(b * OB, OB)],
            out.at[pl.ds(ooff(t), OB)],
            sem_o[b],
        )

    fire_coords(jnp.int32(0), 0)
    fire_coords(jnp.int32(1), 1)
    fill(jnp.int32(0), 0)

    def pair(ip, carry):
        t0 = ip * 2
        fill(t0 + 1, 1)
        drain(t0, 0)

        @pl.when(t0 + 2 < NT)
        def _():
            fill(t0 + 2, 0)

        drain(t0 + 1, 1)
        return carry

    lax.fori_loop(0, NT // 2, pair, jnp.int32(0))
    for b in range(2):
        pltpu.make_async_copy(
            out_v.at[pl.ds(b * OB, OB)],
            out.at[pl.ds(0, OB)],
            sem_o[b],
        ).wait()


_mesh = plsc.VectorSubcoreMesh(
    core_axis_name="c", subcore_axis_name="s", num_cores=NC, num_subcores=NS
)

_run = pl.kernel(
    _sc_body,
    out_type=jax.ShapeDtypeStruct((N * P * C // 2,), jnp.int32),
    mesh=_mesh,
    scratch_types=[
        pltpu.VMEM((2 * CB,), jnp.float32),
        pltpu.VMEM((2 * NROW,), jnp.int32),
        pltpu.VMEM((2 * NROW,), jnp.float32),
        pltpu.VMEM((2 * NROW, C // 2), jnp.int32),
        pltpu.VMEM((2 * OB,), jnp.int32),
        pltpu.SemaphoreType.DMA,
        pltpu.SemaphoreType.DMA,
        pltpu.SemaphoreType.DMA,
        pltpu.SemaphoreType.DMA,
        pltpu.SemaphoreType.DMA,
        pltpu.SemaphoreType.DMA,
    ],
    compiler_params=pltpu.CompilerParams(use_tc_tiling_on_sc=False),
)


def kernel(input, grid):
    vol = jax.lax.bitcast_convert_type(
        jnp.swapaxes(input.reshape(N, C, NVOX), 1, 2)
        .astype(jnp.bfloat16)
        .reshape(N * NVOX, C // 2, 2),
        jnp.int32,
    )
    coords = (
        grid.reshape(N, NW, NCHUNK, BC, 4)
        .transpose(0, 1, 2, 4, 3)
        .reshape(-1)
    )
    out_i32 = _run(vol, coords)
    out_bf = jax.lax.bitcast_convert_type(
        out_i32.reshape(N, P, C // 2), jnp.bfloat16
    ).reshape(N, P, C)
    return jnp.swapaxes(out_bf, 1, 2).astype(jnp.float32)


# speedup vs baseline: 2.5294x; 1.6067x over previous
import jax
import jax.numpy as jnp
import numpy as np
from jax import lax
from jax.experimental import pallas as pl
from jax.experimental.pallas import tpu as pltpu
from jax.experimental.pallas import tpu_sc as plsc

N, C = 2, 32
X, Y, U, V = 16, 16, 32, 32
P = 262144
NVOX = X * Y * U * V
NC, NS = 2, 16
NW = NC * NS
PW = P // NW
BC = 64
NCHUNK = PW // BC
NT = N * NCHUNK
NG = BC // 16
NROW = 16 * BC
NDMA = NROW // 128
CB = 4 * BC
OB = BC * C


def _sc_body(vol, coords, out, coords_v, idx_v, w_v, rows_v, out_v,
             sem_c0, sem_c1, sem_g0, sem_g1, sem_o0, sem_o1):
    wid = lax.axis_index("s") * NC + lax.axis_index("c")
    sem_c = (sem_c0, sem_c1)
    sem_g = (sem_g0, sem_g1)
    sem_o = (sem_o0, sem_o1)

    def coff(t):
        return (((t >> 7) * NW + wid) * NCHUNK + (t & (NCHUNK - 1))) * CB

    def ooff(t):
        return ((t >> 7) * P + wid * PW + (t & (NCHUNK - 1)) * BC) * C

    def fire_coords(t, b):
        pltpu.async_copy(
            coords.at[pl.ds(coff(t), CB)],
            coords_v.at[pl.ds(b * CB, CB)],
            sem_c[b],
        )

    def fill(t, b):
        pltpu.make_async_copy(
            coords.at[pl.ds(0, CB)],
            coords_v.at[pl.ds(b * CB, CB)],
            sem_c[b],
        ).wait()
        nbase = (t >> 7) * NVOX
        for g in range(NG):
            cbb = b * CB
            xs = coords_v[pl.ds(cbb + 0 * BC + g * 16, 16)]
            ys = coords_v[pl.ds(cbb + 1 * BC + g * 16, 16)]
            us = coords_v[pl.ds(cbb + 2 * BC + g * 16, 16)]
            vs = coords_v[pl.ds(cbb + 3 * BC + g * 16, 16)]
            px = ((xs + 1.0) * 0.5) * float(X - 1)
            py = ((ys + 1.0) * 0.5) * float(Y - 1)
            pu = ((us + 1.0) * 0.5) * float(U - 1)
            pv = ((vs + 1.0) * 0.5) * float(V - 1)
            ix = jnp.clip(px.astype(jnp.int32), 0, X - 2)
            iy = jnp.clip(py.astype(jnp.int32), 0, Y - 2)
            iu = jnp.clip(pu.astype(jnp.int32), 0, U - 2)
            iv = jnp.clip(pv.astype(jnp.int32), 0, V - 2)
            fx = px - ix.astype(jnp.float32)
            fy = py - iy.astype(jnp.float32)
            fu = pu - iu.astype(jnp.float32)
            fv = pv - iv.astype(jnp.float32)
            base_i = (ix << 14) + (iy << 10) + (iu << 5) + iv + nbase
            wx = (1.0 - fx, fx)
            wy = (1.0 - fy, fy)
            wu = (1.0 - fu, fu)
            wv_ = (1.0 - fv, fv)
            wxy = [wx[a] * wy[bb] for bb in range(2) for a in range(2)]
            wuv = [wu[a] * wv_[bb] for bb in range(2) for a in range(2)]
            for k in range(16):
                dx, dy = (k >> 0) & 1, (k >> 1) & 1
                du, dv = (k >> 2) & 1, (k >> 3) & 1
                off = dx * 16384 + dy * 1024 + du * 32 + dv
                wk = wxy[dy * 2 + dx] * wuv[dv * 2 + du]
                pos = b * NROW + k * BC + g * 16
                idx_v[pl.ds(pos, 16)] = base_i + off
                w_v[pl.ds(pos, 16)] = wk
        for j in range(NDMA):
            pltpu.async_copy(
                vol.at[idx_v.at[pl.ds(b * NROW + j * 128, 128)]],
                rows_v.at[pl.ds(b * NROW + j * 128, 128), :],
                sem_g[b],
            )

        @pl.when(t + 2 < NT)
        def _():
            fire_coords(t + 2, b)

    def drain(t, b):
        @pl.when(t >= 2)
        def _():
            pltpu.make_async_copy(
                out_v.at[pl.ds(b * OB, OB)],
                out.at[pl.ds(0, OB)],
                sem_o[b],
            ).wait()

        pltpu.make_async_copy(
            vol.at[pl.ds(0, NROW), :],
            rows_v.at[pl.ds(b * NROW, NROW), :],
            sem_g[b],
        ).wait()

        def grp(g2, c2):
            gb = g2 * 16
            wv = [
                w_v[pl.ds(b * NROW + k * BC + gb, 16)] for k in range(16)
            ]
            for l in range(16):
                p = b * NROW + gb + l
                a0 = wv[0][l] * rows_v[p, pl.ds(0, 16)]
                a1 = wv[0][l] * rows_v[p, pl.ds(16, 16)]
                for k in range(1, 16):
                    a0 = a0 + wv[k][l] * rows_v[k * BC + p, pl.ds(0, 16)]
                    a1 = a1 + wv[k][l] * rows_v[k * BC + p, pl.ds(16, 16)]
                op = b * OB + (gb + l) * 32
                out_v[pl.ds(op, 16)] = a0
                out_v[pl.ds(op + 16, 16)] = a1
            return c2

        lax.fori_loop(0, NG, grp, jnp.int32(0))
        pltpu.async_copy(
            out_v.at[pl.ds(b * OB, OB)],
            out.at[pl.ds(ooff(t), OB)],
            sem_o[b],
        )

    fire_coords(jnp.int32(0), 0)
    fire_coords(jnp.int32(1), 1)
    fill(jnp.int32(0), 0)

    def pair(ip, carry):
        t0 = ip * 2
        fill(t0 + 1, 1)
        drain(t0, 0)

        @pl.when(t0 + 2 < NT)
        def _():
            fill(t0 + 2, 0)

        drain(t0 + 1, 1)
        return carry

    lax.fori_loop(0, NT // 2, pair, jnp.int32(0))
    for b in range(2):
        pltpu.make_async_copy(
            out_v.at[pl.ds(b * OB, OB)],
            out.at[pl.ds(0, OB)],
            sem_o[b],
        ).wait()


_mesh = plsc.VectorSubcoreMesh(
    core_axis_name="c", subcore_axis_name="s", num_cores=NC, num_subcores=NS
)

_run = pl.kernel(
    _sc_body,
    out_type=jax.ShapeDtypeStruct((N * P * C,), jnp.float32),
    mesh=_mesh,
    scratch_types=[
        pltpu.VMEM((2 * CB,), jnp.float32),
        pltpu.VMEM((2 * NROW,), jnp.int32),
        pltpu.VMEM((2 * NROW,), jnp.float32),
        pltpu.VMEM((2 * NROW, C), jnp.float32),
        pltpu.VMEM((2 * OB,), jnp.float32),
        pltpu.SemaphoreType.DMA,
        pltpu.SemaphoreType.DMA,
        pltpu.SemaphoreType.DMA,
        pltpu.SemaphoreType.DMA,
        pltpu.SemaphoreType.DMA,
        pltpu.SemaphoreType.DMA,
    ],
    compiler_params=pltpu.CompilerParams(use_tc_tiling_on_sc=False),
)


def kernel(input, grid):
    vol = jnp.swapaxes(input.reshape(N, C, NVOX), 1, 2).reshape(N * NVOX, C)
    coords = (
        grid.reshape(N, NW, NCHUNK, BC, 4)
        .transpose(0, 1, 2, 4, 3)
        .reshape(-1)
    )
    out_flat = _run(vol, coords)
    return jnp.swapaxes(out_flat.reshape(N, P, C), 1, 2)
